# baseline (speedup 1.0000x reference)
import functools

import jax
import jax.numpy as jnp
from jax import lax
from jax.experimental import pallas as pl
from jax.experimental.pallas import tpu as pltpu

_EPS = 1e-5
STATS_BS = 8
CONV_BS = 8


def _stats_kernel(x_ref, o_ref):
    x = x_ref[...]
    s = jnp.sum(x, axis=0)
    q = jnp.sum(x * x, axis=0)
    o_ref[0] = jnp.concatenate(
        [jnp.sum(s, axis=1, keepdims=True),
         jnp.sum(q, axis=1, keepdims=True)], axis=1)


def _bn_partials(x, bs):
    B, C, L = x.shape
    nb = B // bs
    return pl.pallas_call(
        _stats_kernel,
        out_shape=jax.ShapeDtypeStruct((nb, C, 2), jnp.float32),
        grid=(nb,),
        in_specs=[pl.BlockSpec((bs, C, L), lambda i: (i, 0, 0))],
        out_specs=pl.BlockSpec((1, C, 2), lambda i: (i, 0, 0)),
        compiler_params=pltpu.CompilerParams(
            dimension_semantics=("parallel",),
            vmem_limit_bytes=96 << 20),
    )(x)


def _coefs(parts, gamma, beta, inv_n):
    s = jnp.sum(parts, axis=0)
    mean = s[:, 0:1] * inv_n
    var = jnp.maximum(s[:, 1:2] * inv_n - mean * mean, 0.0)
    a = gamma * lax.rsqrt(var + _EPS)
    return a, beta - mean * a


def _cbn_conv_one(x, a, b, scale, shift, w_ref, pad_ref, j,
                  dil, lo, hi, n_taps):
    C, L = x.shape
    y = jnp.maximum(scale * (x * a + b) + shift, 0.0)
    if lo:
        pad_ref[j, :, 0:lo] = jnp.zeros((C, lo), jnp.bfloat16)
    if hi:
        pad_ref[j, :, lo + L:lo + L + hi] = jnp.zeros((C, hi), jnp.bfloat16)
    pad_ref[j, :, lo:lo + L] = y.astype(jnp.bfloat16)
    cols = jnp.concatenate(
        [pad_ref[j, :, k * dil:k * dil + L] for k in range(n_taps)], axis=0)
    return jnp.dot(w_ref[...], cols, preferred_element_type=jnp.float32)


def _conv1_kernel(x_ref, p_ref, g_ref, bt_ref, m_ref, w_ref, b_ref,
                  h_ref, hp_ref, pad_ref, *, dil, lo, hi, n_taps, inv_n, bs):
    a, b = _coefs(p_ref[...], g_ref[...], bt_ref[...], inv_n)
    ss = None
    for j in range(bs):
        conv = _cbn_conv_one(x_ref[j], a, b, m_ref[j, 0:1, :], m_ref[j, 1:2, :],
                             w_ref, pad_ref, j, dil, lo, hi, n_taps)
        hv = conv + b_ref[...]
        h_ref[j] = hv.astype(jnp.bfloat16)
        sj = jnp.concatenate(
            [jnp.sum(hv, axis=1, keepdims=True),
             jnp.sum(hv * hv, axis=1, keepdims=True)], axis=1)
        ss = sj if ss is None else ss + sj
    hp_ref[0] = ss


def _conv2_kernel(h_ref, p_ref, g_ref, bt_ref, m_ref, w_ref, b_ref, skip_ref,
                  o_ref, pad_ref, *, dil, lo, hi, n_taps, inv_n, bs):
    a, b = _coefs(p_ref[...], g_ref[...], bt_ref[...], inv_n)
    for j in range(bs):
        conv = _cbn_conv_one(h_ref[j].astype(jnp.float32), a, b,
                             m_ref[j, 0:1, :], m_ref[j, 1:2, :],
                             w_ref, pad_ref, j, dil, lo, hi, n_taps)
        o_ref[j] = conv + b_ref[...] + skip_ref[j]


def _cbn_relu_conv(x, parts, gamma_c, beta_c, mods3, w_t, bias_c, dil,
                   skip=None, out_dtype=jnp.float32, emit_stats=False, bs=CONV_BS):
    B, C, L = x.shape
    c_out, kc = w_t.shape
    n_taps = kc // C
    total = dil * (n_taps - 1)
    lo = total // 2
    hi = total - lo
    nb = parts.shape[0]
    inv_n = 1.0 / float(B * L)

    in_specs = [
        pl.BlockSpec((bs, C, L), lambda bi: (bi, 0, 0)),
        pl.BlockSpec((nb, C, 2), lambda bi: (0, 0, 0)),
        pl.BlockSpec((C, 1), lambda bi: (0, 0)),
        pl.BlockSpec((C, 1), lambda bi: (0, 0)),
        pl.BlockSpec((bs, 2, L), lambda bi: (bi, 0, 0)),
        pl.BlockSpec((c_out, kc), lambda bi: (0, 0)),
        pl.BlockSpec((c_out, 1), lambda bi: (0, 0)),
    ]
    args = [x, parts, gamma_c, beta_c, mods3, w_t, bias_c]

    out_shape = [jax.ShapeDtypeStruct((B, c_out, L), out_dtype)]
    out_specs = [pl.BlockSpec((bs, c_out, L), lambda bi: (bi, 0, 0))]
    if emit_stats:
        kfn = functools.partial(_conv1_kernel, dil=dil, lo=lo, hi=hi,
                                n_taps=n_taps, inv_n=inv_n, bs=bs)
        out_shape.append(jax.ShapeDtypeStruct((B // bs, c_out, 2), jnp.float32))
        out_specs.append(pl.BlockSpec((1, c_out, 2), lambda bi: (bi, 0, 0)))
    else:
        kfn = functools.partial(_conv2_kernel, dil=dil, lo=lo, hi=hi,
                                n_taps=n_taps, inv_n=inv_n, bs=bs)
        in_specs.append(pl.BlockSpec((bs, c_out, L), lambda bi: (bi, 0, 0)))
        args.append(skip)

    return pl.pallas_call(
        kfn,
        out_shape=out_shape,
        grid=(B // bs,),
        in_specs=in_specs,
        out_specs=out_specs,
        scratch_shapes=[pltpu.VMEM((bs, C, L + total), jnp.bfloat16)],
        compiler_params=pltpu.CompilerParams(
            dimension_semantics=("parallel",),
            vmem_limit_bytes=96 << 20),
    )(*args)


@jax.jit
def kernel(x, z, w_fused_t, b_fused, gamma_r, beta_r, w1_flat, b1_r, w2_flat, b2_r):
    B, C, L = x.shape

    mods = (jnp.dot(z, w_fused_t, precision=lax.Precision.HIGHEST)
            + b_fused)
    mods3 = mods.reshape(B, 2, L)

    gamma_c = gamma_r.reshape(C, 1)
    beta_c = beta_r.reshape(C, 1)
    w1_t = jnp.transpose(w1_flat).astype(jnp.bfloat16)
    w2_t = jnp.transpose(w2_flat).astype(jnp.bfloat16)
    b1_c = b1_r.reshape(-1, 1)
    b2_c = b2_r.reshape(-1, 1)

    parts1 = _bn_partials(x, bs=STATS_BS)
    h, parts2 = _cbn_relu_conv(x, parts1, gamma_c, beta_c, mods3, w1_t, b1_c,
                               dil=1, out_dtype=jnp.bfloat16, emit_stats=True,
                               bs=16)
    (out,) = _cbn_relu_conv(h, parts2, gamma_c, beta_c, mods3, w2_t, b2_c,
                            dil=2, skip=x)
    return out


# speedup vs baseline: 1.0282x; 1.0282x over previous
import functools

import jax
import jax.numpy as jnp
from jax import lax
from jax.experimental import pallas as pl
from jax.experimental.pallas import tpu as pltpu

_EPS = 1e-5
STATS_BS = 8
CONV_BS = 8


def _stats_kernel(x_ref, o_ref):
    x = x_ref[...]
    s = jnp.sum(x, axis=0)
    q = jnp.sum(x * x, axis=0)
    o_ref[0] = jnp.concatenate(
        [jnp.sum(s, axis=1, keepdims=True),
         jnp.sum(q, axis=1, keepdims=True)], axis=1)


def _bn_partials(x, bs):
    B, C, L = x.shape
    nb = B // bs
    return pl.pallas_call(
        _stats_kernel,
        out_shape=jax.ShapeDtypeStruct((nb, C, 2), jnp.float32),
        grid=(nb,),
        in_specs=[pl.BlockSpec((bs, C, L), lambda i: (i, 0, 0))],
        out_specs=pl.BlockSpec((1, C, 2), lambda i: (i, 0, 0)),
        compiler_params=pltpu.CompilerParams(
            dimension_semantics=("parallel",),
            vmem_limit_bytes=96 << 20),
    )(x)


def _coefs(parts, gamma, beta, inv_n):
    s = jnp.sum(parts, axis=0)
    mean = s[:, 0:1] * inv_n
    var = jnp.maximum(s[:, 1:2] * inv_n - mean * mean, 0.0)
    a = gamma * lax.rsqrt(var + _EPS)
    return a, beta - mean * a


def _cbn_conv_one(x, a, b, scale, shift, w_ref, pad_ref, j,
                  dil, lo, hi, n_taps):
    C, L = x.shape
    y = jnp.maximum(scale * (x * a + b) + shift, 0.0)
    if lo:
        pad_ref[j, :, 0:lo] = jnp.zeros((C, lo), jnp.bfloat16)
    if hi:
        pad_ref[j, :, lo + L:lo + L + hi] = jnp.zeros((C, hi), jnp.bfloat16)
    pad_ref[j, :, lo:lo + L] = y.astype(jnp.bfloat16)
    cols = jnp.concatenate(
        [pad_ref[j, :, k * dil:k * dil + L] for k in range(n_taps)], axis=0)
    return jnp.dot(w_ref[...], cols, preferred_element_type=jnp.float32)


def _conv1_kernel(x_ref, p_ref, g_ref, bt_ref, m_ref, w_ref, b_ref,
                  h_ref, hp_ref, pad_ref, *, dil, lo, hi, n_taps, inv_n, bs):
    a, b = _coefs(p_ref[...], g_ref[...], bt_ref[...], inv_n)
    ss = None
    for j in range(bs):
        conv = _cbn_conv_one(x_ref[j], a, b, m_ref[j, 0:1, :], m_ref[j, 1:2, :],
                             w_ref, pad_ref, j, dil, lo, hi, n_taps)
        hv = conv + b_ref[...]
        h_ref[j] = hv.astype(jnp.bfloat16)
        sj = jnp.concatenate(
            [jnp.sum(hv, axis=1, keepdims=True),
             jnp.sum(hv * hv, axis=1, keepdims=True)], axis=1)
        ss = sj if ss is None else ss + sj
    hp_ref[0] = ss


def _conv2_kernel(h_ref, p_ref, g_ref, bt_ref, m_ref, w_ref, b_ref, skip_ref,
                  o_ref, pad_ref, *, dil, lo, hi, n_taps, inv_n, bs):
    a, b = _coefs(p_ref[...], g_ref[...], bt_ref[...], inv_n)
    for j in range(bs):
        conv = _cbn_conv_one(h_ref[j].astype(jnp.float32), a, b,
                             m_ref[j, 0:1, :], m_ref[j, 1:2, :],
                             w_ref, pad_ref, j, dil, lo, hi, n_taps)
        o_ref[j] = conv + b_ref[...] + skip_ref[j]


def _cbn_relu_conv(x, parts, gamma_c, beta_c, mods3, w_t, bias_c, dil,
                   skip=None, out_dtype=jnp.float32, emit_stats=False, bs=CONV_BS):
    B, C, L = x.shape
    c_out, kc = w_t.shape
    n_taps = kc // C
    total = dil * (n_taps - 1)
    lo = total // 2
    hi = total - lo
    nb = parts.shape[0]
    inv_n = 1.0 / float(B * L)

    in_specs = [
        pl.BlockSpec((bs, C, L), lambda bi: (bi, 0, 0)),
        pl.BlockSpec((nb, C, 2), lambda bi: (0, 0, 0)),
        pl.BlockSpec((C, 1), lambda bi: (0, 0)),
        pl.BlockSpec((C, 1), lambda bi: (0, 0)),
        pl.BlockSpec((bs, 2, L), lambda bi: (bi, 0, 0)),
        pl.BlockSpec((c_out, kc), lambda bi: (0, 0)),
        pl.BlockSpec((c_out, 1), lambda bi: (0, 0)),
    ]
    args = [x, parts, gamma_c, beta_c, mods3, w_t, bias_c]

    out_shape = [jax.ShapeDtypeStruct((B, c_out, L), out_dtype)]
    out_specs = [pl.BlockSpec((bs, c_out, L), lambda bi: (bi, 0, 0))]
    if emit_stats:
        kfn = functools.partial(_conv1_kernel, dil=dil, lo=lo, hi=hi,
                                n_taps=n_taps, inv_n=inv_n, bs=bs)
        out_shape.append(jax.ShapeDtypeStruct((B // bs, c_out, 2), jnp.float32))
        out_specs.append(pl.BlockSpec((1, c_out, 2), lambda bi: (bi, 0, 0)))
    else:
        kfn = functools.partial(_conv2_kernel, dil=dil, lo=lo, hi=hi,
                                n_taps=n_taps, inv_n=inv_n, bs=bs)
        in_specs.append(pl.BlockSpec((bs, c_out, L), lambda bi: (bi, 0, 0)))
        args.append(skip)

    return pl.pallas_call(
        kfn,
        out_shape=out_shape,
        grid=(B // bs,),
        in_specs=in_specs,
        out_specs=out_specs,
        scratch_shapes=[pltpu.VMEM((bs, C, L + total), jnp.bfloat16)],
        compiler_params=pltpu.CompilerParams(
            dimension_semantics=("parallel",),
            vmem_limit_bytes=96 << 20),
    )(*args)


@jax.jit
def kernel(x, z, w_fused_t, b_fused, gamma_r, beta_r, w1_flat, b1_r, w2_flat, b2_r):
    B, C, L = x.shape

    mods = (jnp.dot(z, w_fused_t, precision=lax.Precision.HIGHEST)
            + b_fused)
    mods3 = mods.reshape(B, 2, L)

    gamma_c = gamma_r.reshape(C, 1)
    beta_c = beta_r.reshape(C, 1)
    w1_t = jnp.transpose(w1_flat).astype(jnp.bfloat16)
    w2_t = jnp.transpose(w2_flat).astype(jnp.bfloat16)
    b1_c = b1_r.reshape(-1, 1)
    b2_c = b2_r.reshape(-1, 1)

    parts1 = _bn_partials(x, bs=STATS_BS)
    h, parts2 = _cbn_relu_conv(x, parts1, gamma_c, beta_c, mods3, w1_t, b1_c,
                               dil=1, out_dtype=jnp.bfloat16, emit_stats=True)
    (out,) = _cbn_relu_conv(h, parts2, gamma_c, beta_c, mods3, w2_t, b2_c,
                            dil=2, skip=x)
    return out
